# Initial kernel scaffold; baseline (speedup 1.0000x reference)
import jax
import jax.numpy as jnp
from jax.experimental import pallas as pl


def kernel(laplacian_indices, laplacian_values, inputs, weight, bias):
    raise NotImplementedError("write your pallas kernel here")


# speedup vs baseline: 1.6325x; 1.6325x over previous
import functools

import jax
import jax.numpy as jnp
from jax import lax
from jax.experimental import pallas as pl
from jax.experimental.pallas import tpu as pltpu
from jax.experimental.pallas import tpu_sc as plsc

_V = 10000
_E = 320000
_B = 8
_FIN = 128
_K = 3
_FOUT = 128

_NC = 2
_NS = 16
_LANES = 16

_NB = 128
_EPT = _E // _NS
_NFULL = _EPT // _NB
_TAIL = _EPT - _NFULL * _NB
_RPT = 624
_RCH = 104
_NRCH = _RPT // _RCH
_REM = _V - _NS * _RPT
_REMBASE = _NS * _RPT
_PPC = _B // _NC


def _cheb_body(x0, rows, cols, vals, x1, x2,
               acc, obuf, xbuf,
               gbuf, idxg, idxs, valv,
               gbuf_t, idxg_t, idxs_t, valv_t, sem):
    c = lax.axis_index("c")
    s = lax.axis_index("s")
    ebase = s * _EPT

    z16 = jnp.zeros((_LANES,), jnp.float32)

    def _zero_obuf():
        def _zrow(r, carry):
            for j in range(_FIN // _LANES):
                obuf[r, pl.ds(j * _LANES, _LANES)] = z16
            return carry

        lax.fori_loop(0, _RCH, _zrow, 0)

    def _batch(src_hbm, poff, base, n, g, ig, isr, vv):
        pltpu.sync_copy(cols.at[pl.ds(base, n)], ig)
        pltpu.sync_copy(rows.at[pl.ds(base, n)], isr)
        pltpu.sync_copy(vals.at[pl.ds(base, n)], vv)

        def _mkidx(k, carry):
            sl = pl.ds(k * _LANES, _LANES)
            ig[sl] = ig[sl] + poff
            return carry

        lax.fori_loop(0, n // _LANES, _mkidx, 0)

        pltpu.async_copy(src_hbm.at[ig], g, sem).wait()

        def _scale(grp, carry):
            v16 = vv[pl.ds(grp * _LANES, _LANES)]
            for l in range(_LANES):
                e = grp * _LANES + l
                v = v16[l]
                for j in range(_FIN // _LANES):
                    sl = pl.ds(j * _LANES, _LANES)
                    g[e, sl] = g[e, sl] * v
            return carry

        lax.fori_loop(0, n // _LANES, _scale, 0)

        pltpu.sync_copy(g, acc.at[isr], add=True)

    def _accumulate(src_hbm, poff):
        def _full(i, carry):
            _batch(src_hbm, poff, ebase + i * _NB, _NB, gbuf, idxg, idxs, valv)
            return carry

        lax.fori_loop(0, _NFULL, _full, 0)
        _batch(src_hbm, poff, ebase + _NFULL * _NB, _TAIL,
               gbuf_t, idxg_t, idxs_t, valv_t)

    def _zero_acc():
        _zero_obuf()
        for ci in range(_NRCH):
            pltpu.sync_copy(obuf, acc.at[pl.ds(s * _RPT + ci * _RCH, _RCH)])

        @pl.when(s == 0)
        def _():
            pltpu.sync_copy(obuf.at[pl.ds(0, _REM)],
                            acc.at[pl.ds(_REMBASE, _REM)])

    def _combine(nrows):
        def _comb(r, carry2):
            for j in range(_FIN // _LANES):
                sl = pl.ds(j * _LANES, _LANES)
                obuf[r, sl] = obuf[r, sl] * 2.0 - xbuf[r, sl]
            return carry2

        lax.fori_loop(0, nrows, _comb, 0)

    def _panel(q, carry):
        poff = (c * _PPC + q) * _V

        _zero_acc()
        plsc.subcore_barrier()
        _accumulate(x0, poff)
        plsc.subcore_barrier()
        for ci in range(_NRCH):
            r0 = s * _RPT + ci * _RCH
            pltpu.sync_copy(acc.at[pl.ds(r0, _RCH)],
                            x1.at[pl.ds(poff + r0, _RCH)])

        @pl.when(s == 0)
        def _():
            pltpu.sync_copy(acc.at[pl.ds(_REMBASE, _REM)],
                            x1.at[pl.ds(poff + _REMBASE, _REM)])

        plsc.subcore_barrier()

        _zero_acc()
        plsc.subcore_barrier()
        _accumulate(x1, poff)
        plsc.subcore_barrier()
        for ci in range(_NRCH):
            r0 = s * _RPT + ci * _RCH
            pltpu.sync_copy(acc.at[pl.ds(r0, _RCH)], obuf)
            pltpu.sync_copy(x0.at[pl.ds(poff + r0, _RCH)], xbuf)
            _combine(_RCH)
            pltpu.sync_copy(obuf, x2.at[pl.ds(poff + r0, _RCH)])

        @pl.when(s == 0)
        def _():
            pltpu.sync_copy(acc.at[pl.ds(_REMBASE, _REM)],
                            obuf.at[pl.ds(0, _REM)])
            pltpu.sync_copy(x0.at[pl.ds(poff + _REMBASE, _REM)],
                            xbuf.at[pl.ds(0, _REM)])
            _combine(_REM)
            pltpu.sync_copy(obuf.at[pl.ds(0, _REM)],
                            x2.at[pl.ds(poff + _REMBASE, _REM)])

        plsc.subcore_barrier()
        return carry

    lax.fori_loop(0, _PPC, _panel, 0)


@functools.cache
def _build_cheb_sc():
  return pl.kernel(
    _cheb_body,
    out_type=(jax.ShapeDtypeStruct((_B * _V, _FIN), jnp.float32),
              jax.ShapeDtypeStruct((_B * _V, _FIN), jnp.float32)),
    mesh=plsc.VectorSubcoreMesh(core_axis_name="c", subcore_axis_name="s",
                                num_cores=_NC, num_subcores=_NS),
    scratch_types=[
        pltpu.VMEM_SHARED((_V, _FIN), jnp.float32),
        pltpu.VMEM((_RCH, _FIN), jnp.float32),
        pltpu.VMEM((_RCH, _FIN), jnp.float32),
        pltpu.VMEM((_NB, _FIN), jnp.float32),
        pltpu.VMEM((_NB,), jnp.int32),
        pltpu.VMEM((_NB,), jnp.int32),
        pltpu.VMEM((_NB,), jnp.float32),
        pltpu.VMEM((_TAIL, _FIN), jnp.float32),
        pltpu.VMEM((_TAIL,), jnp.int32),
        pltpu.VMEM((_TAIL,), jnp.int32),
        pltpu.VMEM((_TAIL,), jnp.float32),
        pltpu.SemaphoreType.DMA,
    ],
  )


_RB = 2000


def _mm_body(x0b, x1b, x2b, w0, w1, w2, bb, ob):
    acc = jnp.dot(x0b[...], w0[...], preferred_element_type=jnp.float32)
    acc = acc + jnp.dot(x1b[...], w1[...], preferred_element_type=jnp.float32)
    acc = acc + jnp.dot(x2b[...], w2[...], preferred_element_type=jnp.float32)
    ob[...] = acc + bb[...]


def _dense(x0, x1, x2, w0, w1, w2, bias2d):
    nblk = (_B * _V) // _RB
    row_spec = pl.BlockSpec((_RB, _FIN), lambda i: (i, 0))
    full_w = pl.BlockSpec((_FIN, _FOUT), lambda i: (0, 0))
    return pl.pallas_call(
        _mm_body,
        grid=(nblk,),
        in_specs=[row_spec, row_spec, row_spec, full_w, full_w, full_w,
                  pl.BlockSpec((1, _FOUT), lambda i: (0, 0))],
        out_specs=pl.BlockSpec((_RB, _FOUT), lambda i: (i, 0)),
        out_shape=jax.ShapeDtypeStruct((_B * _V, _FOUT), jnp.float32),
    )(x0, x1, x2, w0, w1, w2, bias2d)


def kernel(laplacian_indices, laplacian_values, inputs, weight, bias):
    rows = laplacian_indices[0]
    cols = laplacian_indices[1]
    x0 = inputs.reshape(_B * _V, _FIN)
    x1, x2 = _build_cheb_sc()(x0, rows, cols, laplacian_values)
    w0 = weight[:, 0, :]
    w1 = weight[:, 1, :]
    w2 = weight[:, 2, :]
    out = _dense(x0, x1, x2, w0, w1, w2, bias.reshape(1, _FOUT))
    return out.reshape(_B, _V, _FOUT)


# speedup vs baseline: 3.4729x; 2.1274x over previous
import functools

import jax
import jax.numpy as jnp
from jax import lax
from jax.experimental import pallas as pl
from jax.experimental.pallas import tpu as pltpu
from jax.experimental.pallas import tpu_sc as plsc

_V = 10000
_E = 320000
_B = 8
_FIN = 128
_K = 3
_FOUT = 128

_NC = 2
_NS = 16
_LANES = 16

_NB = 80
_EPT = _E // _NS
_BPT = _EPT // _NB
_RPT = 624
_RCH = 104
_NRCH = _RPT // _RCH
_REM = _V - _NS * _RPT
_REMBASE = _NS * _RPT
_PPC = _B // _NC


def _cheb_body(x0, rows, cols, vals, x1, x2,
               acc, obuf, xbuf,
               g0, g1, c0, c1, r0, r1, v0, v1,
               sg0, sg1, sc0, sc1, srv0, srv1):
    c = lax.axis_index("c")
    s = lax.axis_index("s")
    ebase = s * _EPT

    _G = (g0, g1)
    _C = (c0, c1)
    _R = (r0, r1)
    _VV = (v0, v1)
    _SG = (sg0, sg1)
    _SC = (sc0, sc1)
    _SRV = (srv0, srv1)

    z16 = jnp.zeros((_LANES,), jnp.float32)

    def _zero_obuf():
        def _zrow(r, carry):
            for j in range(_FIN // _LANES):
                obuf[r, pl.ds(j * _LANES, _LANES)] = z16
            return carry

        lax.fori_loop(0, _RCH, _zrow, 0)


    def _issue_cols(j, sl):
        pltpu.async_copy(cols.at[pl.ds(ebase + j * _NB, _NB)],
                         _C[sl], _SC[sl])

    def _wait_cols(sl):
        pltpu.make_async_copy(cols.at[pl.ds(0, _NB)], _C[sl], _SC[sl]).wait()

    def _issue_rv(j, sl):
        pltpu.async_copy(rows.at[pl.ds(ebase + j * _NB, _NB)],
                         _R[sl], _SRV[sl])
        pltpu.async_copy(vals.at[pl.ds(ebase + j * _NB, _NB)],
                         _VV[sl], _SRV[sl])

    def _wait_rv(sl):
        pltpu.make_async_copy(rows.at[pl.ds(0, _NB)], _R[sl], _SRV[sl]).wait()
        pltpu.make_async_copy(vals.at[pl.ds(0, _NB)], _VV[sl], _SRV[sl]).wait()

    def _issue_gather(src_hbm, poff, sl):
        pltpu.async_copy(src_hbm.at[pl.ds(poff, _V)].at[_C[sl]],
                         _G[sl], _SG[sl])

    def _wait_gather(src_hbm, poff, sl):
        pltpu.make_async_copy(src_hbm.at[pl.ds(poff, _V)].at[_C[sl]],
                              _G[sl], _SG[sl]).wait()

    def _scale(sl):
        g = _G[sl]
        vv = _VV[sl]

        def _grp(grp, carry):
            v16 = vv[pl.ds(grp * _LANES, _LANES)]
            for l in range(_LANES):
                e = grp * _LANES + l
                v = v16[l]
                for m in range(_FIN // _LANES):
                    sl2 = pl.ds(m * _LANES, _LANES)
                    g[e, sl2] = g[e, sl2] * v
            return carry

        lax.fori_loop(0, _NB // _LANES, _grp, 0)

    def _step(j, sl, src_hbm, poff):
        other = 1 - sl
        _wait_rv(sl)
        _wait_gather(src_hbm, poff, sl)

        @pl.when(j + 1 < _BPT)
        def _():
            _wait_cols(other)
            _issue_gather(src_hbm, poff, other)
            _issue_rv(j + 1, other)

        @pl.when(j + 2 < _BPT)
        def _():
            _issue_cols(j + 2, sl)

        _scale(sl)
        pltpu.sync_copy(_G[sl], acc.at[_R[sl]], add=True)

    def _accumulate(src_hbm, poff):
        _issue_cols(0, 0)
        _wait_cols(0)
        _issue_gather(src_hbm, poff, 0)
        _issue_rv(0, 0)
        _issue_cols(1, 1)

        def _pair(g_, carry):
            _step(2 * g_, 0, src_hbm, poff)
            _step(2 * g_ + 1, 1, src_hbm, poff)
            return carry

        lax.fori_loop(0, _BPT // 2, _pair, 0)

    def _zero_acc():
        _zero_obuf()
        for ci in range(_NRCH):
            pltpu.sync_copy(obuf, acc.at[pl.ds(s * _RPT + ci * _RCH, _RCH)])

        @pl.when(s == 0)
        def _():
            pltpu.sync_copy(obuf.at[pl.ds(0, _REM)],
                            acc.at[pl.ds(_REMBASE, _REM)])

    def _combine(nrows):
        def _comb(r, carry2):
            for j in range(_FIN // _LANES):
                sl = pl.ds(j * _LANES, _LANES)
                obuf[r, sl] = obuf[r, sl] * 2.0 - xbuf[r, sl]
            return carry2

        lax.fori_loop(0, nrows, _comb, 0)

    def _panel(q, carry):
        poff = (c * _PPC + q) * _V

        _zero_acc()
        plsc.subcore_barrier()
        _accumulate(x0, poff)
        plsc.subcore_barrier()
        for ci in range(_NRCH):
            r0 = s * _RPT + ci * _RCH
            pltpu.sync_copy(acc.at[pl.ds(r0, _RCH)],
                            x1.at[pl.ds(poff + r0, _RCH)])

        @pl.when(s == 0)
        def _():
            pltpu.sync_copy(acc.at[pl.ds(_REMBASE, _REM)],
                            x1.at[pl.ds(poff + _REMBASE, _REM)])

        plsc.subcore_barrier()

        _zero_acc()
        plsc.subcore_barrier()
        _accumulate(x1, poff)
        plsc.subcore_barrier()
        for ci in range(_NRCH):
            r0 = s * _RPT + ci * _RCH
            pltpu.sync_copy(acc.at[pl.ds(r0, _RCH)], obuf)
            pltpu.sync_copy(x0.at[pl.ds(poff + r0, _RCH)], xbuf)
            _combine(_RCH)
            pltpu.sync_copy(obuf, x2.at[pl.ds(poff + r0, _RCH)])

        @pl.when(s == 0)
        def _():
            pltpu.sync_copy(acc.at[pl.ds(_REMBASE, _REM)],
                            obuf.at[pl.ds(0, _REM)])
            pltpu.sync_copy(x0.at[pl.ds(poff + _REMBASE, _REM)],
                            xbuf.at[pl.ds(0, _REM)])
            _combine(_REM)
            pltpu.sync_copy(obuf.at[pl.ds(0, _REM)],
                            x2.at[pl.ds(poff + _REMBASE, _REM)])

        plsc.subcore_barrier()
        return carry

    lax.fori_loop(0, _PPC, _panel, 0)


@functools.cache
def _build_cheb_sc():
  return pl.kernel(
    _cheb_body,
    out_type=(jax.ShapeDtypeStruct((_B * _V, _FIN), jnp.float32),
              jax.ShapeDtypeStruct((_B * _V, _FIN), jnp.float32)),
    mesh=plsc.VectorSubcoreMesh(core_axis_name="c", subcore_axis_name="s",
                                num_cores=_NC, num_subcores=_NS),
    scratch_types=[
        pltpu.VMEM_SHARED((_V, _FIN), jnp.float32),
        pltpu.VMEM((_RCH, _FIN), jnp.float32),
        pltpu.VMEM((_RCH, _FIN), jnp.float32),
        pltpu.VMEM((_NB, _FIN), jnp.float32),
        pltpu.VMEM((_NB, _FIN), jnp.float32),
        pltpu.VMEM((_NB,), jnp.int32),
        pltpu.VMEM((_NB,), jnp.int32),
        pltpu.VMEM((_NB,), jnp.int32),
        pltpu.VMEM((_NB,), jnp.int32),
        pltpu.VMEM((_NB,), jnp.float32),
        pltpu.VMEM((_NB,), jnp.float32),
        pltpu.SemaphoreType.DMA,
        pltpu.SemaphoreType.DMA,
        pltpu.SemaphoreType.DMA,
        pltpu.SemaphoreType.DMA,
        pltpu.SemaphoreType.DMA,
        pltpu.SemaphoreType.DMA,
    ],
  )


_RB = 2000


def _mm_body(x0b, x1b, x2b, w0, w1, w2, bb, ob):
    acc = jnp.dot(x0b[...], w0[...], preferred_element_type=jnp.float32)
    acc = acc + jnp.dot(x1b[...], w1[...], preferred_element_type=jnp.float32)
    acc = acc + jnp.dot(x2b[...], w2[...], preferred_element_type=jnp.float32)
    ob[...] = acc + bb[...]


def _dense(x0, x1, x2, w0, w1, w2, bias2d):
    nblk = (_B * _V) // _RB
    row_spec = pl.BlockSpec((_RB, _FIN), lambda i: (i, 0))
    full_w = pl.BlockSpec((_FIN, _FOUT), lambda i: (0, 0))
    return pl.pallas_call(
        _mm_body,
        grid=(nblk,),
        in_specs=[row_spec, row_spec, row_spec, full_w, full_w, full_w,
                  pl.BlockSpec((1, _FOUT), lambda i: (0, 0))],
        out_specs=pl.BlockSpec((_RB, _FOUT), lambda i: (i, 0)),
        out_shape=jax.ShapeDtypeStruct((_B * _V, _FOUT), jnp.float32),
    )(x0, x1, x2, w0, w1, w2, bias2d)


def kernel(laplacian_indices, laplacian_values, inputs, weight, bias):
    rows = laplacian_indices[0]
    cols = laplacian_indices[1]
    x0 = inputs.reshape(_B * _V, _FIN)
    x1, x2 = _build_cheb_sc()(x0, rows, cols, laplacian_values)
    w0 = weight[:, 0, :]
    w1 = weight[:, 1, :]
    w2 = weight[:, 2, :]
    out = _dense(x0, x1, x2, w0, w1, w2, bias.reshape(1, _FOUT))
    return out.reshape(_B, _V, _FOUT)
